# baseline (speedup 1.0000x reference)
import functools

import jax
import jax.numpy as jnp
from jax import lax
from jax.experimental import pallas as pl
from jax.experimental.pallas import tpu as pltpu
from jax.experimental.pallas import tpu_sc as plsc

VOCAB = 1000000
DIM = 64
BATCH = 16384
NEG = 5

NC = 2
NS = 16
NW = NC * NS
L = 16

BPW = BATCH // NW
CH = 64
NCH = BPW // CH
NGRP = CH // L
UNROLL = 4
W = 2 * DIM


def _softplus(x):
    z = jnp.exp(-jnp.abs(x))
    w = z / (z + 2.0)
    w2 = w * w
    p = 1.0 + w2 * (1.0 / 3.0 + w2 * (1.0 / 5.0 + w2 * (1.0 / 7.0 + w2 * (1.0 / 9.0))))
    return jnp.maximum(x, 0.0) + 2.0 * w * p


def _body(pos_u_hbm, pos_v_hbm, neg_hbm, tbl_hbm, out_hbm,
          idx_u, idx_v, idx_n,
          ru0, rv0, rn0, ru1, rv1, rn1, loss_v,
          su0, sv0, sn0, su1, sv1, sn1):
    bufs = ((ru0, rv0, rn0), (ru1, rv1, rn1))
    sems = ((su0, sv0, sn0), (su1, sv1, sn1))

    c_id = lax.axis_index("c")
    s_id = lax.axis_index("s")
    wid = s_id * NC + c_id
    base = wid * BPW

    pltpu.sync_copy(pos_u_hbm.at[pl.ds(base, BPW)], idx_u)
    pltpu.sync_copy(pos_v_hbm.at[pl.ds(base, BPW)], idx_v)
    pltpu.sync_copy(neg_hbm.at[pl.ds(base * NEG, BPW * NEG)], idx_n)

    lane = lax.iota(jnp.int32, L)
    loss = jnp.zeros((L,), jnp.float32)

    def start_fetch(c, s):
        ru, rv, rn = bufs[s]
        semu, semv, semn = sems[s]
        cps = [
            pltpu.async_copy(tbl_hbm.at[idx_u.at[pl.ds(c * CH, CH)]], ru, semu),
            pltpu.async_copy(tbl_hbm.at[idx_v.at[pl.ds(c * CH, CH)]], rv, semv),
        ]
        for j in range(NEG):
            cps.append(pltpu.async_copy(
                tbl_hbm.at[idx_n.at[pl.ds(c * CH * NEG + j * CH, CH)]],
                rn.at[pl.ds(j * CH, CH)], semn))
        return cps

    pend = {0: start_fetch(0, 0)}

    for c in range(NCH):
        s = c % 2
        if c + 1 < NCH:
            pend[c + 1] = start_fetch(c + 1, 1 - s)
        for cp in pend.pop(c):
            cp.wait()
        ru, rv, rn = bufs[s]

        def group(g, loss):
            rb = g * L + lane
            rbn = [rb * NEG + j for j in range(NEG)]

            def dstep(t, accs):
                ap, a0, a1, a2, a3, a4 = accs
                for q in range(UNROLL):
                    d = t * UNROLL + q
                    dc = jnp.broadcast_to(d, (L,))
                    dv = jnp.broadcast_to(d + DIM, (L,))
                    uc = plsc.load_gather(ru, [rb, dc])
                    vc = plsc.load_gather(rv, [rb, dv])
                    ap = ap + uc * vc
                    a0 = a0 + plsc.load_gather(rn, [rbn[0], dv]) * uc
                    a1 = a1 + plsc.load_gather(rn, [rbn[1], dv]) * uc
                    a2 = a2 + plsc.load_gather(rn, [rbn[2], dv]) * uc
                    a3 = a3 + plsc.load_gather(rn, [rbn[3], dv]) * uc
                    a4 = a4 + plsc.load_gather(rn, [rbn[4], dv]) * uc
                return ap, a0, a1, a2, a3, a4

            z = jnp.zeros((L,), jnp.float32)
            ap, a0, a1, a2, a3, a4 = lax.fori_loop(
                0, DIM // UNROLL, dstep, (z,) * 6)

            loss = loss + _softplus(-jnp.clip(ap, -10.0, 10.0))
            for t in (a0, a1, a2, a3, a4):
                loss = loss + _softplus(jnp.clip(t, -10.0, 10.0))
            return loss

        loss = lax.fori_loop(0, NGRP, group, loss)

    loss_v[...] = loss
    pltpu.sync_copy(loss_v, out_hbm.at[wid])


_mesh = plsc.VectorSubcoreMesh(core_axis_name="c", subcore_axis_name="s")

_sgns = functools.partial(
    pl.kernel,
    mesh=_mesh,
    compiler_params=pltpu.CompilerParams(
        needs_layout_passes=False, use_tc_tiling_on_sc=True),
    out_type=jax.ShapeDtypeStruct((NW, L), jnp.float32),
    scratch_types=[
        pltpu.VMEM((BPW,), jnp.int32),
        pltpu.VMEM((BPW,), jnp.int32),
        pltpu.VMEM((BPW * NEG,), jnp.int32),
        pltpu.VMEM((CH, W), jnp.float32),
        pltpu.VMEM((CH, W), jnp.float32),
        pltpu.VMEM((CH * NEG, W), jnp.float32),
        pltpu.VMEM((CH, W), jnp.float32),
        pltpu.VMEM((CH, W), jnp.float32),
        pltpu.VMEM((CH * NEG, W), jnp.float32),
        pltpu.VMEM((L,), jnp.float32),
    ] + [pltpu.SemaphoreType.DMA] * 6,
)(_body)


@jax.jit
def kernel(pos_u, pos_v, neg_v, u_emb, v_emb):
    pos_u = pos_u.astype(jnp.int32)
    pos_v = pos_v.astype(jnp.int32)
    neg_f = neg_v.reshape(-1).astype(jnp.int32)
    tbl = jnp.concatenate([u_emb, v_emb], axis=1)
    parts = _sgns(pos_u, pos_v, neg_f, tbl)
    return jnp.sum(parts) * (1.0 / BATCH)
